# baseline (speedup 1.0000x reference)
import jax
import jax.numpy as jnp
from jax import lax
from jax.experimental import pallas as pl
from jax.experimental.pallas import tpu as pltpu
from jax.experimental.pallas import tpu_sc as plsc

NC = 2
NS = 16
NW = NC * NS
L = 16

B = 4096
F_NUM = 13
F_CAT = 26
CARD1 = 1001
D = 128
T_TOK = F_NUM + F_CAT

B_PER_W = B // NW
NBUF = 6
LOOKAHEAD = 4


def _tokenizer_body(tab_hbm, xcat_hbm, out_hbm, idx_v, bufs, gsem, ssem):
    wid = lax.axis_index("s") * NC + lax.axis_index("c")
    b0 = wid * B_PER_W

    pltpu.sync_copy(xcat_hbm.at[:, pl.ds(b0, B_PER_W)], idx_v)

    iota = lax.iota(jnp.int32, L)

    def _gidx_body(f, carry):
        off = f * CARD1 + 1
        for v in range(B_PER_W // L):
            sl = pl.ds(v * L, L)
            idx_v[f, sl] = idx_v[f, sl] + off
        return carry

    lax.fori_loop(0, F_CAT, _gidx_body, 0)

    def _fire_gather(p):
        j = p % NBUF
        return pltpu.async_copy(tab_hbm.at[idx_v.at[p]], bufs[j], gsem[j])

    def _fire_scatter(p):
        j = p % NBUF
        return pltpu.async_copy(
            bufs[j].reshape(B_PER_W, 1, D),
            out_hbm.at[pl.ds(b0, B_PER_W), pl.ds(F_NUM + p, 1)],
            ssem[j])

    gh = [None] * F_CAT
    sh = [None] * F_CAT
    for p in range(min(LOOKAHEAD, F_CAT)):
        gh[p] = _fire_gather(p)
    for p in range(F_CAT):
        gh[p].wait()
        sh[p] = _fire_scatter(p)
        q = p + LOOKAHEAD
        if q < F_CAT:
            d = q - NBUF
            if d >= 0:
                sh[d].wait()
            gh[q] = _fire_gather(q)
    for p in range(F_CAT - NBUF, F_CAT):
        if p >= 0 and sh[p] is not None:
            sh[p].wait()


def _build_sc_call():
    mesh = plsc.VectorSubcoreMesh(
        core_axis_name="c", subcore_axis_name="s",
        num_cores=NC, num_subcores=NS)
    scratch = [
        pltpu.VMEM((F_CAT, B_PER_W), jnp.int32),
        [pltpu.VMEM((B_PER_W, D), jnp.float32) for _ in range(NBUF)],
        [pltpu.SemaphoreType.DMA for _ in range(NBUF)],
        [pltpu.SemaphoreType.DMA for _ in range(NBUF)],
    ]
    return pl.kernel(
        _tokenizer_body,
        out_type=jax.ShapeDtypeStruct((B, T_TOK, D), jnp.float32),
        mesh=mesh,
        scratch_types=scratch,
        name="feature_tokenizer_sc",
    )


_SC_CALL = _build_sc_call()

NB = 512


def _tc_body(xn_ref, w_ref, b_ref, out_ref):
    out_ref[...] = (xn_ref[...][:, :, None] * w_ref[...][None, :, :]
                    + b_ref[...][None, :, :])


_TC_CALL = pl.pallas_call(
    _tc_body,
    grid=(B // NB,),
    in_specs=[
        pl.BlockSpec((NB, F_NUM), lambda i: (i, 0)),
        pl.BlockSpec((F_NUM, D), lambda i: (0, 0)),
        pl.BlockSpec((F_NUM, D), lambda i: (0, 0)),
    ],
    out_specs=pl.BlockSpec((NB, F_NUM, D), lambda i: (i, 0, 0)),
    out_shape=jax.ShapeDtypeStruct((B, F_NUM, D), jnp.float32),
)


def kernel(x_cat, x_num, W_num, b_num, tables):
    xcat_t = x_cat.astype(jnp.int32).T
    tab = tables.reshape(F_CAT * CARD1, D)
    cat_full = _SC_CALL(tab, xcat_t)
    num_tok = _TC_CALL(x_num, W_num, b_num)
    return lax.dynamic_update_slice(cat_full, num_tok, (0, 0, 0))


# speedup vs baseline: 2.6508x; 2.6508x over previous
import jax
import jax.numpy as jnp
from jax import lax
from jax.experimental import pallas as pl
from jax.experimental.pallas import tpu as pltpu
from jax.experimental.pallas import tpu_sc as plsc

NC = 2
NS = 16
NW = NC * NS
L = 16

B = 4096
F_NUM = 13
F_CAT = 26
CARD1 = 1001
D = 128
T_TOK = F_NUM + F_CAT

B_PER_W = B // NW
NBUF = 4
NNUM = 2
LOOKAHEAD = 3


def _tokenizer_body(tab_hbm, xcat_hbm, xnum_hbm, w_hbm, b_hbm, out_hbm,
                    idx_v, xnum_v, w_v, b_v, bufs, nbufs, gsem, ssem, nsem):
    wid = lax.axis_index("s") * NC + lax.axis_index("c")
    b0 = wid * B_PER_W

    pltpu.sync_copy(xcat_hbm.at[:, pl.ds(b0, B_PER_W)], idx_v)
    pltpu.sync_copy(xnum_hbm.at[:, pl.ds(b0, B_PER_W)], xnum_v)
    pltpu.sync_copy(w_hbm, w_v)
    pltpu.sync_copy(b_hbm, b_v)

    iota = lax.iota(jnp.int32, L)

    def _gidx_body(f, carry):
        off = f * CARD1 + 1
        for v in range(B_PER_W // L):
            sl = pl.ds(v * L, L)
            idx_v[f, sl] = idx_v[f, sl] + off
        return carry

    lax.fori_loop(0, F_CAT, _gidx_body, 0)

    def _fire_gather(p):
        j = p % NBUF
        return pltpu.async_copy(tab_hbm.at[idx_v.at[p]], bufs[j], gsem[j])

    def _fire_scatter(p):
        j = p % NBUF
        return pltpu.async_copy(
            bufs[j].reshape(B_PER_W, 1, D),
            out_hbm.at[pl.ds(b0, B_PER_W), pl.ds(F_NUM + p, 1)],
            ssem[j])

    def _compute_numeric(i):
        j = i % NNUM
        wv = [w_v[i, pl.ds(dv * L, L)] for dv in range(D // L)]
        bv = [b_v[i, pl.ds(dv * L, L)] for dv in range(D // L)]

        def _num_body(v, carry, wv=wv, bv=bv, i=i, j=j):
            off = pl.multiple_of(v * L, L)
            xv = xnum_v[i, pl.ds(off, L)]
            for l in range(L):
                xs = xv[l]
                bl = v * L + l
                for dv in range(D // L):
                    nbufs[j][bl, pl.ds(dv * L, L)] = xs * wv[dv] + bv[dv]
            return carry

        lax.fori_loop(0, B_PER_W // L, _num_body, 0)
        return pltpu.async_copy(
            nbufs[j].reshape(B_PER_W, 1, D),
            out_hbm.at[pl.ds(b0, B_PER_W), pl.ds(i, 1)], nsem[j])

    gh = [None] * F_CAT
    sh = [None] * F_CAT
    nh = [None] * F_NUM
    for p in range(LOOKAHEAD):
        gh[p] = _fire_gather(p)
    for p in range(F_CAT):
        q = p + LOOKAHEAD
        if q < F_CAT:
            d = q - NBUF
            if d >= 0:
                sh[d].wait()
            gh[q] = _fire_gather(q)
        if p < F_NUM:
            if p >= NNUM:
                nh[p - NNUM].wait()
            nh[p] = _compute_numeric(p)
        gh[p].wait()
        sh[p] = _fire_scatter(p)
    for p in range(F_CAT - NBUF, F_CAT):
        sh[p].wait()
    for i in range(F_NUM - NNUM, F_NUM):
        nh[i].wait()


def _build_sc_call():
    mesh = plsc.VectorSubcoreMesh(
        core_axis_name="c", subcore_axis_name="s",
        num_cores=NC, num_subcores=NS)
    scratch = [
        pltpu.VMEM((F_CAT, B_PER_W), jnp.int32),
        pltpu.VMEM((F_NUM, B_PER_W), jnp.float32),
        pltpu.VMEM((F_NUM, D), jnp.float32),
        pltpu.VMEM((F_NUM, D), jnp.float32),
        [pltpu.VMEM((B_PER_W, D), jnp.float32) for _ in range(NBUF)],
        [pltpu.VMEM((B_PER_W, D), jnp.float32) for _ in range(NNUM)],
        [pltpu.SemaphoreType.DMA for _ in range(NBUF)],
        [pltpu.SemaphoreType.DMA for _ in range(NBUF)],
        [pltpu.SemaphoreType.DMA for _ in range(NNUM)],
    ]
    return pl.kernel(
        _tokenizer_body,
        out_type=jax.ShapeDtypeStruct((B, T_TOK, D), jnp.float32),
        mesh=mesh,
        scratch_types=scratch,
        name="feature_tokenizer_sc",
    )


_SC_CALL = _build_sc_call()


def kernel(x_cat, x_num, W_num, b_num, tables):
    xcat_t = x_cat.astype(jnp.int32).T
    xnum_t = x_num.T
    tab = tables.reshape(F_CAT * CARD1, D)
    return _SC_CALL(tab, xcat_t, xnum_t, W_num, b_num)


# speedup vs baseline: 2.6628x; 1.0045x over previous
import jax
import jax.numpy as jnp
from jax import lax
from jax.experimental import pallas as pl
from jax.experimental.pallas import tpu as pltpu
from jax.experimental.pallas import tpu_sc as plsc

NC = 2
NS = 16
NW = NC * NS
L = 16

B = 4096
F_NUM = 13
F_CAT = 26
CARD1 = 1001
D = 128
T_TOK = F_NUM + F_CAT

B_PER_W = B // NW
NBUF = 5
NNUM = 2
LOOKAHEAD = 4


def _tokenizer_body(tab_hbm, xcat_hbm, xnum_hbm, w_hbm, b_hbm, out_hbm,
                    idx_v, xnum_v, w_v, b_v, bufs, nbufs, gsem, ssem, nsem):
    wid = lax.axis_index("s") * NC + lax.axis_index("c")
    b0 = wid * B_PER_W

    pltpu.sync_copy(xcat_hbm.at[:, pl.ds(b0, B_PER_W)], idx_v)
    pltpu.sync_copy(xnum_hbm.at[:, pl.ds(b0, B_PER_W)], xnum_v)
    pltpu.sync_copy(w_hbm, w_v)
    pltpu.sync_copy(b_hbm, b_v)

    iota = lax.iota(jnp.int32, L)

    def _gidx_body(f, carry):
        off = f * CARD1 + 1
        for v in range(B_PER_W // L):
            sl = pl.ds(v * L, L)
            idx_v[f, sl] = idx_v[f, sl] + off
        return carry

    lax.fori_loop(0, F_CAT, _gidx_body, 0)

    def _fire_gather(p):
        j = p % NBUF
        return pltpu.async_copy(tab_hbm.at[idx_v.at[p]], bufs[j], gsem[j])

    def _fire_scatter(p):
        j = p % NBUF
        return pltpu.async_copy(
            bufs[j].reshape(B_PER_W, 1, D),
            out_hbm.at[pl.ds(b0, B_PER_W), pl.ds(F_NUM + p, 1)],
            ssem[j])

    def _compute_numeric(i):
        j = i % NNUM
        wv = [w_v[i, pl.ds(dv * L, L)] for dv in range(D // L)]
        bv = [b_v[i, pl.ds(dv * L, L)] for dv in range(D // L)]

        def _num_body(v, carry, wv=wv, bv=bv, i=i, j=j):
            off = pl.multiple_of(v * L, L)
            xv = xnum_v[i, pl.ds(off, L)]
            for l in range(L):
                xs = xv[l]
                bl = v * L + l
                for dv in range(D // L):
                    nbufs[j][bl, pl.ds(dv * L, L)] = xs * wv[dv] + bv[dv]
            return carry

        lax.fori_loop(0, B_PER_W // L, _num_body, 0)
        return pltpu.async_copy(
            nbufs[j].reshape(B_PER_W, 1, D),
            out_hbm.at[pl.ds(b0, B_PER_W), pl.ds(i, 1)], nsem[j])

    gh = [None] * F_CAT
    sh = [None] * F_CAT
    nh = [None] * F_NUM
    for p in range(LOOKAHEAD):
        gh[p] = _fire_gather(p)
    for p in range(F_CAT):
        q = p + LOOKAHEAD
        if q < F_CAT:
            d = q - NBUF
            if d >= 0:
                sh[d].wait()
            gh[q] = _fire_gather(q)
        if p < F_NUM:
            if p >= NNUM:
                nh[p - NNUM].wait()
            nh[p] = _compute_numeric(p)
        gh[p].wait()
        sh[p] = _fire_scatter(p)
    for p in range(F_CAT - NBUF, F_CAT):
        sh[p].wait()
    for i in range(F_NUM - NNUM, F_NUM):
        nh[i].wait()


def _build_sc_call():
    mesh = plsc.VectorSubcoreMesh(
        core_axis_name="c", subcore_axis_name="s",
        num_cores=NC, num_subcores=NS)
    scratch = [
        pltpu.VMEM((F_CAT, B_PER_W), jnp.int32),
        pltpu.VMEM((F_NUM, B_PER_W), jnp.float32),
        pltpu.VMEM((F_NUM, D), jnp.float32),
        pltpu.VMEM((F_NUM, D), jnp.float32),
        [pltpu.VMEM((B_PER_W, D), jnp.float32) for _ in range(NBUF)],
        [pltpu.VMEM((B_PER_W, D), jnp.float32) for _ in range(NNUM)],
        [pltpu.SemaphoreType.DMA for _ in range(NBUF)],
        [pltpu.SemaphoreType.DMA for _ in range(NBUF)],
        [pltpu.SemaphoreType.DMA for _ in range(NNUM)],
    ]
    return pl.kernel(
        _tokenizer_body,
        out_type=jax.ShapeDtypeStruct((B, T_TOK, D), jnp.float32),
        mesh=mesh,
        scratch_types=scratch,
        name="feature_tokenizer_sc",
    )


_SC_CALL = _build_sc_call()


def kernel(x_cat, x_num, W_num, b_num, tables):
    xcat_t = x_cat.astype(jnp.int32).T
    xnum_t = x_num.T
    tab = tables.reshape(F_CAT * CARD1, D)
    return _SC_CALL(tab, xcat_t, xnum_t, W_num, b_num)
